# baseline (speedup 1.0000x reference)
import functools

import numpy as np
import jax
import jax.numpy as jnp
from jax import lax
from jax.experimental import pallas as pl
from jax.experimental.pallas import tpu as pltpu, tpu_sc as plsc

_NC = 2
_NS = 16
_NW = _NC * _NS


def _pos_table(seq_len, d):
    j = np.arange(d // 2, dtype=np.float64)
    units = 10000.0 ** (2.0 * j / d)
    p = np.arange(seq_len, dtype=np.float64)[:, None]
    angle = p / units[None, :]
    pos = np.zeros((seq_len, d), dtype=np.float64)
    pos[:, 0::2] = np.sin(angle)
    pos[:, 1::2] = np.cos(angle)
    return jnp.asarray(pos, dtype=jnp.float32)


@functools.lru_cache(maxsize=None)
def _make_sc_kernel(B, S, D):
    assert B % _NW == 0 and D % 16 == 0 and S % 8 == 0
    n_seq_w = B // _NW
    g0 = min(128, S) // 8 * 8
    pieces = [(0, g0)]
    if g0 < S:
        pieces.append((g0, S - g0))
    mesh = plsc.VectorSubcoreMesh(core_axis_name="c", subcore_axis_name="s")

    @functools.partial(
        pl.kernel,
        out_type=jax.ShapeDtypeStruct((B * S, D), jnp.float32),
        mesh=mesh,
        compiler_params=pltpu.CompilerParams(use_tc_tiling_on_sc=False),
        scratch_types=[
            pltpu.VMEM((S,), jnp.int32),
            pltpu.VMEM((S, D), jnp.float32),
            pltpu.VMEM((S, D), jnp.float32),
            pltpu.SemaphoreType.DMA,
        ],
    )
    def k(idx_hbm, table_hbm, pos_hbm, out_hbm, idx_v, rows_v, pos_v, sem):
        wid = lax.axis_index("s") * _NC + lax.axis_index("c")
        pltpu.sync_copy(pos_hbm, pos_v)
        seq0 = wid * n_seq_w

        def seq_body(i, _):
            base = (seq0 + i) * S
            pltpu.sync_copy(idx_hbm.at[pl.ds(base, S)], idx_v)
            cps = [
                pltpu.async_copy(
                    table_hbm.at[idx_v.at[pl.ds(off, n)]],
                    rows_v.at[pl.ds(off, n)],
                    sem,
                )
                for off, n in pieces
            ]
            for cp in cps:
                cp.wait()

            def row_body(r, _):
                for dd in range(D // 16):
                    sl = pl.ds(dd * 16, 16)
                    rows_v[r, sl] = rows_v[r, sl] + pos_v[r, sl]
                return 0

            lax.fori_loop(0, S, row_body, 0)
            pltpu.sync_copy(rows_v, out_hbm.at[pl.ds(base, S)])
            return 0

        lax.fori_loop(0, n_seq_w, seq_body, 0)

    return k


def kernel(x, table):
    B, S = x.shape
    D = table.shape[1]
    pos = _pos_table(S, D)
    idx = x.astype(jnp.int32).reshape(B * S)
    out = _make_sc_kernel(B, S, D)(idx, table, pos)
    return out.reshape(B, S, D)
